# Initial kernel scaffold; baseline (speedup 1.0000x reference)
import jax
import jax.numpy as jnp
from jax.experimental import pallas as pl


def kernel(user_ids, positive_product_ids, negative_product_ids, positive_comment_embeddings, user_table, product_table, W, b):
    raise NotImplementedError("write your pallas kernel here")


# speedup vs baseline: 4.3745x; 4.3745x over previous
import functools

import jax
import jax.numpy as jnp
from jax import lax
from jax.experimental import pallas as pl
from jax.experimental.pallas import tpu as pltpu
from jax.experimental.pallas import tpu_sc as plsc

_B = 16384
_D = 128
_NC = 2
_NS = 16
_NW = _NC * _NS
_CHUNK = 128
_CPW = _B // (_NW * _CHUNK)


def _sc_gather_body(uids, pids, nids, utab, ptab, uout, pout, nout,
                    idx_v, rows_v, sem):
    wid = lax.axis_index("s") * _NC + lax.axis_index("c")
    for ids, tab, out in ((uids, utab, uout), (pids, ptab, pout),
                          (nids, ptab, nout)):
        pltpu.sync_copy(ids.at[wid], idx_v)
        copies = [
            pltpu.async_copy(tab.at[idx_v.at[j]], rows_v.at[j], sem)
            for j in range(_CPW)
        ]
        for c in copies:
            c.wait()
        pltpu.sync_copy(rows_v, out.at[wid])


_sc_gather = functools.partial(
    pl.kernel,
    mesh=plsc.VectorSubcoreMesh(core_axis_name="c", subcore_axis_name="s"),
    out_type=(
        jax.ShapeDtypeStruct((_NW, _CPW, _CHUNK, _D), jnp.float32),
        jax.ShapeDtypeStruct((_NW, _CPW, _CHUNK, _D), jnp.float32),
        jax.ShapeDtypeStruct((_NW, _CPW, _CHUNK, _D), jnp.float32),
    ),
    scratch_types=[
        pltpu.VMEM((_CPW, _CHUNK), jnp.int32),
        pltpu.VMEM((_CPW, _CHUNK, _D), jnp.float32),
        pltpu.SemaphoreType.DMA,
    ],
)(_sc_gather_body)


_BLK = 2048


def _tc_score_body(u_ref, p_ref, c_ref, n_ref, w_ref, b_ref, sp_ref, sn_ref):
    u = u_ref[...]
    p = p_ref[...]
    cm = c_ref[...]
    n = n_ref[...]
    w = w_ref[...]
    w1 = w[:, :_D]
    w2 = w[:, _D:]
    fused = (
        lax.dot_general(p, w1, (((1,), (1,)), ((), ())),
                        preferred_element_type=jnp.float32)
        + lax.dot_general(cm, w2, (((1,), (1,)), ((), ())),
                          preferred_element_type=jnp.float32)
        + b_ref[...]
    )
    sp_ref[...] = jnp.sum(u * fused, axis=1)
    sn_ref[...] = jnp.sum(u * n, axis=1)


def _tc_score(u, p, cm, n, w, b2):
    grid = _B // _BLK
    row_spec = pl.BlockSpec((_BLK, _D), lambda i: (i, 0))
    return pl.pallas_call(
        _tc_score_body,
        grid=(grid,),
        in_specs=[
            row_spec, row_spec, row_spec, row_spec,
            pl.BlockSpec((_D, 2 * _D), lambda i: (0, 0)),
            pl.BlockSpec((1, _D), lambda i: (0, 0)),
        ],
        out_specs=[
            pl.BlockSpec((_BLK,), lambda i: (i,)),
            pl.BlockSpec((_BLK,), lambda i: (i,)),
        ],
        out_shape=[
            jax.ShapeDtypeStruct((_B,), jnp.float32),
            jax.ShapeDtypeStruct((_B,), jnp.float32),
        ],
    )(u, p, cm, n, w, b2)


def kernel(user_ids, positive_product_ids, negative_product_ids,
           positive_comment_embeddings, user_table, product_table, W, b):
    uids = user_ids.reshape(_NW, _CPW, _CHUNK)
    pids = positive_product_ids.reshape(_NW, _CPW, _CHUNK)
    nids = negative_product_ids.reshape(_NW, _CPW, _CHUNK)
    ue, pe, ne = _sc_gather(uids, pids, nids, user_table, product_table)
    ue = ue.reshape(_B, _D)
    pe = pe.reshape(_B, _D)
    ne = ne.reshape(_B, _D)
    score_pos, score_neg = _tc_score(
        ue, pe, positive_comment_embeddings, ne, W, b.reshape(1, _D))
    return (score_pos, score_neg)
